# Initial kernel scaffold; baseline (speedup 1.0000x reference)
import jax
import jax.numpy as jnp
from jax.experimental import pallas as pl


def kernel(proj_pt, dist, idx_before, idx_after, lane_features):
    raise NotImplementedError("write your pallas kernel here")


# speedup vs baseline: 1.6163x; 1.6163x over previous
import jax
import jax.numpy as jnp
from jax import lax
from jax.experimental import pallas as pl
from jax.experimental.pallas import tpu as pltpu

_BLOCK = 1000


def _body(lf_ref, aux_ref, idx_ref, out_ref):
    v = lf_ref[...]
    aux = aux_ref[...]
    idx = idx_ref[...]

    B, W = v.shape

    d = v[:, 4:W] - v[:, 0 : W - 4]
    sq = d * d
    pr = sq[:, 0 : W - 5] + sq[:, 1 : W - 4]
    sp = jnp.sqrt(pr)

    c = lax.broadcasted_iota(jnp.int32, (1, W - 5), 1)
    idx4 = idx * 4
    mask_s = ((c & 3) == 0) & (c < idx4)
    s_base = jnp.sum(jnp.where(mask_s, sp, 0.0), axis=1, keepdims=True)

    c6 = lax.broadcasted_iota(jnp.int32, (1, W), 1)
    xb = jnp.sum(jnp.where(c6 == idx4, v, 0.0), axis=1, keepdims=True)
    yb = jnp.sum(jnp.where(c6 == idx4 + 1, v, 0.0), axis=1, keepdims=True)
    xa = jnp.sum(jnp.where(c6 == idx4 + 4, v, 0.0), axis=1, keepdims=True)
    ya = jnp.sum(jnp.where(c6 == idx4 + 5, v, 0.0), axis=1, keepdims=True)

    vx = xa - xb
    vy = ya - yb
    mag = jnp.sqrt(vx * vx + vy * vy)
    ux = vx / mag
    uy = vy / mag

    px = aux[:, 0:1]
    py = aux[:, 1:2]
    dx = aux[:, 2:3]
    dy = aux[:, 3:4]

    s = s_base + (px - xb) * ux + (py - yb) * uy
    l = dx * uy - dy * ux
    out_ref[...] = jnp.concatenate([s, l], axis=1)


def kernel(proj_pt, dist, idx_before, idx_after, lane_features):
    del idx_after
    N, P, C = lane_features.shape
    lf = lane_features.reshape(N, P * C)
    aux = jnp.concatenate([proj_pt, dist], axis=1)
    idx = idx_before.astype(jnp.int32).reshape(N, 1)

    grid = (N // _BLOCK,)
    return pl.pallas_call(
        _body,
        grid=grid,
        in_specs=[
            pl.BlockSpec((_BLOCK, P * C), lambda i: (i, 0)),
            pl.BlockSpec((_BLOCK, 4), lambda i: (i, 0)),
            pl.BlockSpec((_BLOCK, 1), lambda i: (i, 0)),
        ],
        out_specs=pl.BlockSpec((_BLOCK, 2), lambda i: (i, 0)),
        out_shape=jax.ShapeDtypeStruct((N, 2), jnp.float32),
        compiler_params=pltpu.CompilerParams(
            dimension_semantics=("arbitrary",),
        ),
    )(lf, aux, idx)


# speedup vs baseline: 1.9020x; 1.1767x over previous
import jax
import jax.numpy as jnp
from jax import lax
from jax.experimental import pallas as pl
from jax.experimental.pallas import tpu as pltpu

_BLOCK = 1000


def _body(x_ref, y_ref, aux_ref, idx_ref, out_ref):
    x = x_ref[...]
    y = y_ref[...]
    aux = aux_ref[...]
    idx = idx_ref[...]

    B, P = x.shape

    dxp = x[:, 1:] - x[:, :-1]
    dyp = y[:, 1:] - y[:, :-1]
    sp = jnp.sqrt(dxp * dxp + dyp * dyp)

    k = lax.broadcasted_iota(jnp.int32, (1, P - 1), 1)
    s_base = jnp.sum(jnp.where(k < idx, sp, 0.0), axis=1, keepdims=True)

    c = lax.broadcasted_iota(jnp.int32, (1, P), 1)
    mb = c == idx
    ma = c == idx + 1
    xb = jnp.sum(jnp.where(mb, x, 0.0), axis=1, keepdims=True)
    yb = jnp.sum(jnp.where(mb, y, 0.0), axis=1, keepdims=True)
    xa = jnp.sum(jnp.where(ma, x, 0.0), axis=1, keepdims=True)
    ya = jnp.sum(jnp.where(ma, y, 0.0), axis=1, keepdims=True)

    vx = xa - xb
    vy = ya - yb
    mag = jnp.sqrt(vx * vx + vy * vy)
    ux = vx / mag
    uy = vy / mag

    px = aux[:, 0:1]
    py = aux[:, 1:2]
    dx = aux[:, 2:3]
    dy = aux[:, 3:4]

    s = s_base + (px - xb) * ux + (py - yb) * uy
    l = dx * uy - dy * ux
    out_ref[...] = jnp.concatenate([s, l], axis=1)


def kernel(proj_pt, dist, idx_before, idx_after, lane_features):
    del idx_after
    N, P, C = lane_features.shape
    x = lane_features[:, :, 0]
    y = lane_features[:, :, 1]
    aux = jnp.concatenate([proj_pt, dist], axis=1)
    idx = idx_before.astype(jnp.int32).reshape(N, 1)

    grid = (N // _BLOCK,)
    return pl.pallas_call(
        _body,
        grid=grid,
        in_specs=[
            pl.BlockSpec((_BLOCK, P), lambda i: (i, 0)),
            pl.BlockSpec((_BLOCK, P), lambda i: (i, 0)),
            pl.BlockSpec((_BLOCK, 4), lambda i: (i, 0)),
            pl.BlockSpec((_BLOCK, 1), lambda i: (i, 0)),
        ],
        out_specs=pl.BlockSpec((_BLOCK, 2), lambda i: (i, 0)),
        out_shape=jax.ShapeDtypeStruct((N, 2), jnp.float32),
        compiler_params=pltpu.CompilerParams(
            dimension_semantics=("arbitrary",),
        ),
    )(x, y, aux, idx)
